# baseline (speedup 1.0000x reference)
import functools

import jax
import jax.numpy as jnp
from jax.experimental import pallas as pl
from jax.experimental.pallas import tpu as pltpu

BLK = 128
CUTOFF = 5.0
EPS = 1e-8
PAD_GI = 10**9


_COS_C = (9.9999999228e-01, -4.9999991772e-01, 4.1666524364e-02,
          -1.3887970411e-03, 2.4773424205e-05, -2.7113373272e-07,
          1.7369133674e-09)
_SIN_C = (9.9999970703e-01, -1.6666577217e-01, 8.3325581326e-03,
          -1.9812575931e-04, 2.7040516974e-06, -2.0534265044e-08)
_TWO_PI_HI = 6.28125
_TWO_PI_LO = 1.9353071795864768e-03
_INV_TWO_PI = 0.15915494309189535


def _cos_clamped(y):
    y = jnp.clip(y, -jnp.pi, jnp.pi)
    z = y * y
    c = _COS_C
    r = c[6]
    for k in (5, 4, 3, 2, 1, 0):
        r = r * z + c[k]
    return r


def _sin_reduced(a):
    q = jnp.floor(a * _INV_TWO_PI + 0.5)
    x = (a - q * _TWO_PI_HI) - q * _TWO_PI_LO
    z = x * x
    s = _SIN_C
    r = s[5]
    for k in (4, 3, 2, 1, 0):
        r = r * z + s[k]
    return x * r


def _phi_body(atoms_ref, embT_ref, Ws1_ref, bs1_ref, Ws2b_ref, bs2b_ref, phiT_ref):
    atp = embT_ref.shape[1]
    arow = atoms_ref[...]
    ioa = jax.lax.broadcasted_iota(jnp.int32, (atp, BLK), 0)
    oh = (ioa == arow).astype(jnp.float32)
    s0T = jax.lax.dot_general(embT_ref[...], oh, (((1,), (0,)), ((), ())),
                              preferred_element_type=jnp.float32)
    h = jax.lax.dot_general(Ws1_ref[...], s0T, (((1,), (0,)), ((), ())),
                            preferred_element_type=jnp.float32) + bs1_ref[...]
    h = h * jax.nn.sigmoid(h)
    phiT_ref[...] = jax.lax.dot_general(Ws2b_ref[...], h, (((1,), (0,)), ((), ())),
                                        preferred_element_type=jnp.float32) + bs2b_ref[...]


def _pair_body(bi_s, bj_s, first_s, last_s, act_s,
               pos_i_ref, posT_j_ref, gcol_ref, grow_ref, Wr2_ref, br2_ref,
               kcol_ref, out_ref, acc_ref, tmp_ref, *, nf, num_rbf):
    p = pl.program_id(0)

    @pl.when(first_s[p] == 1)
    def _():
        acc_ref[...] = jnp.zeros_like(acc_ref)

    @pl.when(act_s[p] == 1)
    def _():
        pos_i = pos_i_ref[...]
        posTj = posT_j_ref[...]
        xi, yi, zi = pos_i[:, 0:1], pos_i[:, 1:2], pos_i[:, 2:3]
        xj, yj, zj = posTj[0:1, :], posTj[1:2, :], posTj[2:3, :]
        g = xi * xj + yi * yj + zi * zj
        pn_i = xi * xi + yi * yi + zi * zi
        pn_j = xj * xj + yj * yj + zj * zj
        d2 = jnp.maximum(pn_i + pn_j - 2.0 * g, 0.0)
        dist = jnp.sqrt(d2)
        m = jnp.logical_and(gcol_ref[...] == grow_ref[...],
                            dist < CUTOFF).astype(jnp.float32)
        nr = jnp.sqrt(d2 + 1e-12)
        inv = 1.0 / (nr + EPS)
        tmp_ref[0 * BLK:1 * BLK, :] = (jnp.pi / CUTOFF) * nr
        tmp_ref[1 * BLK:2 * BLK, :] = inv
        tmp_ref[2 * BLK:3 * BLK, :] = m
        tmp_ref[3 * BLK:4 * BLK, :] = m * ((xi - xj) * inv)
        tmp_ref[4 * BLK:5 * BLK, :] = m * ((yi - yj) * inv)
        tmp_ref[5 * BLK:6 * BLK, :] = m * ((zi - zj) * inv)
        rowact = jnp.sum(m, axis=1, keepdims=True)
        idxc = jax.lax.broadcasted_iota(jnp.int32, (BLK, 1), 0)
        lo = jnp.min(jnp.where(rowact > 0.0, idxc, BLK))
        hi = jnp.max(jnp.where(rowact > 0.0, idxc, -1))
        kvec = kcol_ref[...]
        Wr2 = Wr2_ref[...]
        br2 = br2_ref[...]

        def body(i, carry):
            th = tmp_ref[pl.ds(0 * BLK + i, 1), :]
            iv = tmp_ref[pl.ds(1 * BLK + i, 1), :]
            rbf = _sin_reduced(kvec * th) * iv
            lin = jax.lax.dot_general(Wr2, rbf, (((1,), (0,)), ((), ())),
                                      preferred_element_type=jnp.float32) + br2
            Wp = (0.5 * (_cos_clamped((jnp.pi / CUTOFF) * lin) + 1.0)
                  * (lin < CUTOFF).astype(jnp.float32))
            W2 = Wp[0:nf, :]
            W3 = Wp[nf:2 * nf, :]
            acc_ref[0:nf, :] += W2 * tmp_ref[pl.ds(2 * BLK + i, 1), :]
            acc_ref[nf:2 * nf, :] += W3 * tmp_ref[pl.ds(3 * BLK + i, 1), :]
            acc_ref[2 * nf:3 * nf, :] += W3 * tmp_ref[pl.ds(4 * BLK + i, 1), :]
            acc_ref[3 * nf:4 * nf, :] += W3 * tmp_ref[pl.ds(5 * BLK + i, 1), :]
            return carry

        jax.lax.fori_loop(lo, hi + 1, body, 0)

    @pl.when(last_s[p] == 1)
    def _():
        out_ref[...] = acc_ref[...]


def _mm(a, b, precision=None):
    return jax.lax.dot_general(a, b, (((1,), (0,)), ((), ())),
                               preferred_element_type=jnp.float32,
                               precision=precision)


def _packed_body(bj_s, iw_s, first_s, last_s, act_s, cm_s,
                 posw1_ref, posw2_ref, posj_ref, irel_ref, jl_ref, jlc_ref,
                 valid_ref, Wr2_ref, br2_ref, kcol_ref, out_ref, acc_ref, *, nf):
    p = pl.program_id(0)

    @pl.when(first_s[p] == 1)
    def _():
        acc_ref[...] = jnp.zeros_like(acc_ref)

    @pl.when(act_s[p] == 1)
    def _():
        irel = irel_ref[0]
        jl = jl_ref[0]
        jlc = jlc_ref[...]
        vrow = valid_ref[0]
        ios = jax.lax.broadcasted_iota(jnp.int32, (BLK, BLK), 0)
        iol = jax.lax.broadcasted_iota(jnp.int32, (BLK, BLK), 1)
        Gi1 = (ios == irel).astype(jnp.float32)
        Gi2 = (ios == (irel - BLK)).astype(jnp.float32)
        Gj = (ios == jl).astype(jnp.float32)
        GjT = (jlc == iol).astype(jnp.float32)
        hi = jax.lax.Precision.HIGHEST
        pi3 = (_mm(posw1_ref[...], Gi1, hi)
               + _mm(posw2_ref[...], Gi2, hi))
        pj3 = _mm(posj_ref[...], Gj, hi)
        xi, yi, zi = pi3[0:1, :], pi3[1:2, :], pi3[2:3, :]
        xj, yj, zj = pj3[0:1, :], pj3[1:2, :], pj3[2:3, :]
        dx = xi - xj
        dy = yi - yj
        dz = zi - zj
        d2m = ((xi * xi + yi * yi + zi * zi) + (xj * xj + yj * yj + zj * zj)
               - 2.0 * (xi * xj + yi * yj + zi * zj))
        dist = jnp.sqrt(jnp.maximum(d2m, 0.0))
        m = vrow * (dist < CUTOFF).astype(jnp.float32)
        nr = jnp.sqrt(dx * dx + dy * dy + dz * dz + 1e-12)
        inv = 1.0 / (nr + EPS)
        th = (jnp.pi / CUTOFF) * nr
        rbf = _sin_reduced(kcol_ref[...] * th) * inv
        lin = _mm(Wr2_ref[...], rbf, hi) + br2_ref[...]
        Wp = (0.5 * (_cos_clamped((jnp.pi / CUTOFF) * lin) + 1.0)
              * (lin < CUTOFF).astype(jnp.float32))
        W2 = Wp[0:nf, :]
        W3 = Wp[nf:2 * nf, :]
        tmp = jnp.concatenate(
            [W2 * m, W3 * (m * (dx * inv)), W3 * (m * (dy * inv)),
             W3 * (m * (dz * inv))], axis=0)
        acc_ref[...] += _mm(tmp, GjT, hi)

    @pl.when(last_s[p] == 1)
    def _():
        out_ref[...] = acc_ref[...]


def _update_body(accT_ref, phiT_ref,
                 WUx_ref, bUx_ref, WUy_ref, bUy_ref, WUz_ref, bUz_ref,
                 WVx_ref, bVx_ref, WVy_ref, bVy_ref, WVz_ref, bVz_ref,
                 Wm1_ref, bm1_ref, Wm2_ref, bm2_ref,
                 ds_ref, dv_ref, *, nf):
    phi2 = phiT_ref[0:nf, :]
    phi3 = phiT_ref[nf:2 * nf, :]
    s1T = phi2 * accT_ref[0:nf, :]
    v1x = phi3 * accT_ref[nf:2 * nf, :]
    v1y = phi3 * accT_ref[2 * nf:3 * nf, :]
    v1z = phi3 * accT_ref[3 * nf:4 * nf, :]
    Uvx = _mm(WUx_ref[...], v1x) + bUx_ref[...]
    Uvy = _mm(WUy_ref[...], v1y) + bUy_ref[...]
    Uvz = _mm(WUz_ref[...], v1z) + bUz_ref[...]
    Vvx = _mm(WVx_ref[...], v1x) + bVx_ref[...]
    Vvy = _mm(WVy_ref[...], v1y) + bVy_ref[...]
    Vvz = _mm(WVz_ref[...], v1z) + bVz_ref[...]
    Vn = jnp.sqrt(Vvx * Vvx + Vvy * Vvy + Vvz * Vvz + 1e-12)
    Wm1 = Wm1_ref[...]
    h = _mm(Wm1[:, 0:nf], Vn) + _mm(Wm1[:, nf:2 * nf], s1T) + bm1_ref[...]
    h = h * jax.nn.sigmoid(h)
    mlp = _mm(Wm2_ref[...], h) + bm2_ref[...]
    a_vv = mlp[0:nf, :]
    a_sv = mlp[nf:2 * nf, :]
    a_ss = mlp[2 * nf:3 * nf, :]
    ds_ref[...] = (Uvx * Vvx + Uvy * Vvy + Uvz * Vvz) * a_sv + a_ss
    dv_ref[0:nf, :] = a_vv * Uvx
    dv_ref[nf:2 * nf, :] = a_vv * Uvy
    dv_ref[2 * nf:3 * nf, :] = a_vv * Uvz


def kernel(atoms, atom_positions, graph_indexes, emb, Ws1, bs1, Ws2, bs2, Wr, br,
           Wm1, bm1, Wm2, bm2, WUx, bUx, WUy, bUy, WUz, bUz, WVx, bVx, WVy, bVy,
           WVz, bVz):
    n = atoms.shape[0]
    nf = emb.shape[1]
    num_rbf = Wr.shape[1]
    natype = emb.shape[0]
    atp = ((natype + BLK - 1) // BLK) * BLK
    npad = ((n + BLK - 1) // BLK) * BLK
    nb = npad // BLK
    npair = nb * nb

    f32 = jnp.float32
    pos = jnp.zeros((npad, 3), f32).at[:n].set(atom_positions.astype(f32))
    posT = pos.T
    gi = jnp.full((npad,), PAD_GI, jnp.int32).at[:n].set(
        graph_indexes.astype(jnp.int32))
    gcol = gi[:, None]
    grow = gi[None, :]
    atoms_row = jnp.zeros((1, npad), jnp.int32).at[0, :n].set(
        atoms.astype(jnp.int32))

    embT = jnp.zeros((nf, atp), f32).at[:, :natype].set(emb.T)
    Ws2b = Ws2[nf:3 * nf, :]
    bs2b = bs2[nf:3 * nf][:, None]
    Wr2 = Wr[nf:3 * nf, :]
    br2 = br[nf:3 * nf][:, None]

    phiT = pl.pallas_call(
        _phi_body,
        grid=(nb,),
        in_specs=[
            pl.BlockSpec((1, BLK), lambda b: (0, b)),
            pl.BlockSpec((nf, atp), lambda b: (0, 0)),
            pl.BlockSpec((nf, nf), lambda b: (0, 0)),
            pl.BlockSpec((nf, 1), lambda b: (0, 0)),
            pl.BlockSpec((2 * nf, nf), lambda b: (0, 0)),
            pl.BlockSpec((2 * nf, 1), lambda b: (0, 0)),
        ],
        out_specs=pl.BlockSpec((2 * nf, BLK), lambda b: (0, b)),
        out_shape=jax.ShapeDtypeStruct((2 * nf, npad), f32),
    )(atoms_row, embT, Ws1, bs1[:, None], Ws2b, bs2b)

    gb = gi.reshape(nb, BLK)
    glo = gb[:, 0]
    ghi = gb[:, -1]
    act_mat = (glo[:, None] <= ghi[None, :]) & (glo[None, :] <= ghi[:, None])
    flat = act_mat.T.reshape(-1)
    idx = jnp.nonzero(flat, size=npair, fill_value=0)[0]
    num = jnp.sum(flat.astype(jnp.int32))
    ar = jnp.arange(npair)
    lastidx = idx[num - 1]
    pidx = jnp.where(ar < num, idx, lastidx)
    bi_arr = (pidx % nb).astype(jnp.int32)
    bj_arr = (pidx // nb).astype(jnp.int32)
    act_arr = (ar < num).astype(jnp.int32)
    bj_prev = jnp.concatenate([jnp.array([-1], jnp.int32), bj_arr[:-1]])
    bj_next = jnp.concatenate([bj_arr[1:], jnp.array([-1], jnp.int32)])
    first_arr = ((bj_arr != bj_prev) & (ar < num)).astype(jnp.int32)
    last_arr = (((ar + 1 == num) | (bj_arr != bj_next)) & (ar < num)).astype(
        jnp.int32)

    grid_spec = pltpu.PrefetchScalarGridSpec(
        num_scalar_prefetch=5,
        grid=(npair,),
        in_specs=[
            pl.BlockSpec((BLK, 3), lambda p, bi, bj, f, l, a: (bi[p], 0)),
            pl.BlockSpec((3, BLK), lambda p, bi, bj, f, l, a: (0, bj[p])),
            pl.BlockSpec((BLK, 1), lambda p, bi, bj, f, l, a: (bi[p], 0)),
            pl.BlockSpec((1, BLK), lambda p, bi, bj, f, l, a: (0, bj[p])),
            pl.BlockSpec((2 * nf, num_rbf), lambda p, bi, bj, f, l, a: (0, 0)),
            pl.BlockSpec((2 * nf, 1), lambda p, bi, bj, f, l, a: (0, 0)),
            pl.BlockSpec((num_rbf, 1), lambda p, bi, bj, f, l, a: (0, 0)),
        ],
        out_specs=pl.BlockSpec((4 * nf, BLK), lambda p, bi, bj, f, l, a: (0, bj[p])),
        scratch_shapes=[pltpu.VMEM((4 * nf, BLK), f32),
                        pltpu.VMEM((6 * BLK, BLK), f32)],
    )
    accT_call = pl.pallas_call(
        functools.partial(_pair_body, nf=nf, num_rbf=num_rbf),
        grid_spec=grid_spec,
        out_shape=jax.ShapeDtypeStruct((4 * nf, npad), f32),
    )
    kcol = jnp.arange(1, num_rbf + 1, dtype=f32)[:, None]

    i32 = jnp.int32
    rcap = 4 * npad
    cap = npad // 2
    capl = cap * BLK
    ms = jnp.searchsorted(gi, gi, side='left').astype(i32)
    me = jnp.searchsorted(gi, gi, side='right').astype(i32)
    blk0 = jnp.arange(nb, dtype=i32) * BLK
    ilo = jnp.minimum(ms[blk0], n)
    ihi = jnp.minimum(me[blk0 + BLK - 1], n)
    cnt = jnp.maximum(ihi - ilo, 0)
    rstart = jnp.concatenate([jnp.zeros(1, i32),
                              jnp.cumsum(cnt, dtype=i32)])
    rtot = rstart[nb]
    r = jnp.arange(rcap, dtype=i32)
    rb = jnp.clip(jnp.searchsorted(rstart, r, side='right').astype(i32) - 1,
                  0, nb - 1)
    ri = jnp.clip(ilo[rb] + (r - rstart[rb]), 0, n - 1)
    rjs = jnp.maximum(ms[ri], rb * BLK)
    rje = jnp.minimum(me[ri], (rb + 1) * BLK)
    rL = jnp.where(r < rtot, rje - rjs, 0)
    Sb = jnp.zeros((nb,), i32).at[rb].add(rL)
    Spad = jnp.maximum(((Sb + BLK - 1) // BLK) * BLK, BLK)
    lanebase = jnp.concatenate([jnp.zeros(1, i32),
                                jnp.cumsum(Spad, dtype=i32)])
    chunkstart = lanebase // BLK
    ctot = chunkstart[nb]
    rLex = jnp.cumsum(rL, dtype=i32) - rL
    off_in_block = rLex - rLex[rstart[rb]]
    gls = jnp.where(r < rtot, lanebase[rb] + off_in_block, capl).astype(i32)
    lane = jnp.arange(capl, dtype=i32)
    lb = jnp.clip(jnp.searchsorted(lanebase, lane, side='right').astype(i32) - 1,
                  0, nb - 1)
    lvalid = (lane - lanebase[lb]) < Sb[lb]
    lrun = jnp.clip(jnp.searchsorted(gls, lane, side='right').astype(i32) - 1,
                    0, rcap - 1)
    li = ri[lrun]
    ljl = jnp.clip(rjs[lrun] + (lane - gls[lrun]) - lb * BLK, 0, BLK - 1)
    liM = li.reshape(cap, BLK)
    lvM = lvalid.reshape(cap, BLK)
    minli = jnp.min(jnp.where(lvM, liM, 1 << 30), axis=1)
    iw_c = jnp.clip(minli // BLK, 0, nb - 1).astype(i32)
    irelM = jnp.clip(liM - iw_c[:, None] * BLK, 0, 2 * BLK - 1).astype(i32)
    c = jnp.arange(cap, dtype=i32)
    cb = jnp.clip(jnp.searchsorted(chunkstart, c, side='right').astype(i32) - 1,
                  0, nb - 1)
    actc = c < ctot
    firstc = (actc & (c == chunkstart[cb])).astype(i32)
    lastc = (actc & (c + 1 == chunkstart[cb + 1])).astype(i32)
    lastreal = jnp.maximum(ctot - 1, 0)
    iw_c = jnp.where(actc, iw_c, iw_c[lastreal]).astype(i32)
    cm_c = jnp.minimum(c, lastreal).astype(i32)
    actc_i = actc.astype(i32)
    okpack = (rtot <= rcap) & (lanebase[nb] <= capl)

    posg = jnp.zeros((3, (nb + 1) * BLK), f32).at[:, :npad].set(posT)
    irel3 = irelM.reshape(cap, 1, BLK)
    jl3 = ljl.reshape(cap, 1, BLK)
    jlc2 = ljl.reshape(capl, 1)
    valid3 = lvM.astype(f32).reshape(cap, 1, BLK)

    packed_spec = pltpu.PrefetchScalarGridSpec(
        num_scalar_prefetch=6,
        grid=(cap,),
        in_specs=[
            pl.BlockSpec((3, BLK), lambda p, bj, iw, fi, la, ac, cm: (0, iw[p])),
            pl.BlockSpec((3, BLK), lambda p, bj, iw, fi, la, ac, cm: (0, iw[p] + 1)),
            pl.BlockSpec((3, BLK), lambda p, bj, iw, fi, la, ac, cm: (0, bj[p])),
            pl.BlockSpec((1, 1, BLK), lambda p, bj, iw, fi, la, ac, cm: (cm[p], 0, 0)),
            pl.BlockSpec((1, 1, BLK), lambda p, bj, iw, fi, la, ac, cm: (cm[p], 0, 0)),
            pl.BlockSpec((BLK, 1), lambda p, bj, iw, fi, la, ac, cm: (cm[p], 0)),
            pl.BlockSpec((1, 1, BLK), lambda p, bj, iw, fi, la, ac, cm: (cm[p], 0, 0)),
            pl.BlockSpec((2 * nf, num_rbf), lambda p, bj, iw, fi, la, ac, cm: (0, 0)),
            pl.BlockSpec((2 * nf, 1), lambda p, bj, iw, fi, la, ac, cm: (0, 0)),
            pl.BlockSpec((num_rbf, 1), lambda p, bj, iw, fi, la, ac, cm: (0, 0)),
        ],
        out_specs=pl.BlockSpec((4 * nf, BLK),
                               lambda p, bj, iw, fi, la, ac, cm: (0, bj[p])),
        scratch_shapes=[pltpu.VMEM((4 * nf, BLK), f32)],
    )
    packed_call = pl.pallas_call(
        functools.partial(_packed_body, nf=nf),
        grid_spec=packed_spec,
        out_shape=jax.ShapeDtypeStruct((4 * nf, npad), f32),
    )

    def _run_packed(_):
        return packed_call(bj_arr_p, iw_c, firstc, lastc, actc_i, cm_c,
                           posg, posg, posg, irel3, jl3, jlc2, valid3,
                           Wr2, br2, kcol)

    def _run_tiles(_):
        return accT_call(bi_arr, bj_arr, first_arr, last_arr, act_arr,
                         pos, posT, gcol, grow, Wr2, br2, kcol)

    bj_arr_p = cb
    accT = jax.lax.cond(okpack, _run_packed, _run_tiles, None)

    wspec = pl.BlockSpec((nf, nf), lambda b: (0, 0))
    bspec = pl.BlockSpec((nf, 1), lambda b: (0, 0))
    dsT, dvT = pl.pallas_call(
        functools.partial(_update_body, nf=nf),
        grid=(nb,),
        in_specs=[
            pl.BlockSpec((4 * nf, BLK), lambda b: (0, b)),
            pl.BlockSpec((2 * nf, BLK), lambda b: (0, b)),
            wspec, bspec, wspec, bspec, wspec, bspec,
            wspec, bspec, wspec, bspec, wspec, bspec,
            pl.BlockSpec((nf, 2 * nf), lambda b: (0, 0)),
            bspec,
            pl.BlockSpec((3 * nf, nf), lambda b: (0, 0)),
            pl.BlockSpec((3 * nf, 1), lambda b: (0, 0)),
        ],
        out_specs=[
            pl.BlockSpec((nf, BLK), lambda b: (0, b)),
            pl.BlockSpec((3 * nf, BLK), lambda b: (0, b)),
        ],
        out_shape=[
            jax.ShapeDtypeStruct((nf, npad), f32),
            jax.ShapeDtypeStruct((3 * nf, npad), f32),
        ],
    )(accT, phiT,
      WUx, bUx[:, None], WUy, bUy[:, None], WUz, bUz[:, None],
      WVx, bVx[:, None], WVy, bVy[:, None], WVz, bVz[:, None],
      Wm1, bm1[:, None], Wm2, bm2[:, None])

    delta_s = dsT.T[:n]
    delta_v = jnp.stack([dvT[0:nf, :].T[:n], dvT[nf:2 * nf, :].T[:n],
                         dvT[2 * nf:3 * nf, :].T[:n]], axis=2)
    return delta_s, delta_v


# speedup vs baseline: 40.5791x; 40.5791x over previous
import functools

import jax
import jax.numpy as jnp
from jax.experimental import pallas as pl
from jax.experimental.pallas import tpu as pltpu

BLK = 128
CUTOFF = 5.0
EPS = 1e-8
PAD_GI = 10**9


_COS_C = (9.9999999228e-01, -4.9999991772e-01, 4.1666524364e-02,
          -1.3887970411e-03, 2.4773424205e-05, -2.7113373272e-07,
          1.7369133674e-09)
_SIN_C = (9.9999970703e-01, -1.6666577217e-01, 8.3325581326e-03,
          -1.9812575931e-04, 2.7040516974e-06, -2.0534265044e-08)
_TWO_PI_HI = 6.28125
_TWO_PI_LO = 1.9353071795864768e-03
_INV_TWO_PI = 0.15915494309189535


def _cos_clamped(y):
    y = jnp.clip(y, -jnp.pi, jnp.pi)
    z = y * y
    c = _COS_C
    r = c[6]
    for k in (5, 4, 3, 2, 1, 0):
        r = r * z + c[k]
    return r


def _sin_reduced(a):
    q = jnp.floor(a * _INV_TWO_PI + 0.5)
    x = (a - q * _TWO_PI_HI) - q * _TWO_PI_LO
    z = x * x
    s = _SIN_C
    r = s[5]
    for k in (4, 3, 2, 1, 0):
        r = r * z + s[k]
    return x * r


def _phi_body(atoms_ref, embT_ref, Ws1_ref, bs1_ref, Ws2b_ref, bs2b_ref, phiT_ref):
    atp = embT_ref.shape[1]
    arow = atoms_ref[...]
    ioa = jax.lax.broadcasted_iota(jnp.int32, (atp, BLK), 0)
    oh = (ioa == arow).astype(jnp.float32)
    s0T = jax.lax.dot_general(embT_ref[...], oh, (((1,), (0,)), ((), ())),
                              preferred_element_type=jnp.float32)
    h = jax.lax.dot_general(Ws1_ref[...], s0T, (((1,), (0,)), ((), ())),
                            preferred_element_type=jnp.float32) + bs1_ref[...]
    h = h * jax.nn.sigmoid(h)
    phiT_ref[...] = jax.lax.dot_general(Ws2b_ref[...], h, (((1,), (0,)), ((), ())),
                                        preferred_element_type=jnp.float32) + bs2b_ref[...]


def _pair_body(bi_s, bj_s, first_s, last_s, act_s,
               pos_i_ref, posT_j_ref, gcol_ref, grow_ref, Wr2_ref, br2_ref,
               kcol_ref, out_ref, acc_ref, tmp_ref, *, nf, num_rbf):
    p = pl.program_id(0)

    @pl.when(first_s[p] == 1)
    def _():
        acc_ref[...] = jnp.zeros_like(acc_ref)

    @pl.when(act_s[p] == 1)
    def _():
        pos_i = pos_i_ref[...]
        posTj = posT_j_ref[...]
        xi, yi, zi = pos_i[:, 0:1], pos_i[:, 1:2], pos_i[:, 2:3]
        xj, yj, zj = posTj[0:1, :], posTj[1:2, :], posTj[2:3, :]
        g = xi * xj + yi * yj + zi * zj
        pn_i = xi * xi + yi * yi + zi * zi
        pn_j = xj * xj + yj * yj + zj * zj
        d2 = jnp.maximum(pn_i + pn_j - 2.0 * g, 0.0)
        dist = jnp.sqrt(d2)
        m = jnp.logical_and(gcol_ref[...] == grow_ref[...],
                            dist < CUTOFF).astype(jnp.float32)
        nr = jnp.sqrt(d2 + 1e-12)
        inv = 1.0 / (nr + EPS)
        tmp_ref[0 * BLK:1 * BLK, :] = (jnp.pi / CUTOFF) * nr
        tmp_ref[1 * BLK:2 * BLK, :] = inv
        tmp_ref[2 * BLK:3 * BLK, :] = m
        tmp_ref[3 * BLK:4 * BLK, :] = m * ((xi - xj) * inv)
        tmp_ref[4 * BLK:5 * BLK, :] = m * ((yi - yj) * inv)
        tmp_ref[5 * BLK:6 * BLK, :] = m * ((zi - zj) * inv)
        rowact = jnp.sum(m, axis=1, keepdims=True)
        idxc = jax.lax.broadcasted_iota(jnp.int32, (BLK, 1), 0)
        lo = jnp.min(jnp.where(rowact > 0.0, idxc, BLK))
        hi = jnp.max(jnp.where(rowact > 0.0, idxc, -1))
        kvec = kcol_ref[...]
        Wr2 = Wr2_ref[...]
        br2 = br2_ref[...]

        def body(i, carry):
            th = tmp_ref[pl.ds(0 * BLK + i, 1), :]
            iv = tmp_ref[pl.ds(1 * BLK + i, 1), :]
            rbf = _sin_reduced(kvec * th) * iv
            lin = jax.lax.dot_general(Wr2, rbf, (((1,), (0,)), ((), ())),
                                      preferred_element_type=jnp.float32) + br2
            Wp = (0.5 * (_cos_clamped((jnp.pi / CUTOFF) * lin) + 1.0)
                  * (lin < CUTOFF).astype(jnp.float32))
            W2 = Wp[0:nf, :]
            W3 = Wp[nf:2 * nf, :]
            acc_ref[0:nf, :] += W2 * tmp_ref[pl.ds(2 * BLK + i, 1), :]
            acc_ref[nf:2 * nf, :] += W3 * tmp_ref[pl.ds(3 * BLK + i, 1), :]
            acc_ref[2 * nf:3 * nf, :] += W3 * tmp_ref[pl.ds(4 * BLK + i, 1), :]
            acc_ref[3 * nf:4 * nf, :] += W3 * tmp_ref[pl.ds(5 * BLK + i, 1), :]
            return carry

        jax.lax.fori_loop(lo, hi + 1, body, 0)

    @pl.when(last_s[p] == 1)
    def _():
        out_ref[...] = acc_ref[...]


def _mm(a, b, precision=None):
    return jax.lax.dot_general(a, b, (((1,), (0,)), ((), ())),
                               preferred_element_type=jnp.float32,
                               precision=precision)


def _packed_body(bj_s, iw_s, first_s, last_s, act_s, cm_s,
                 posw1_ref, posw2_ref, posj_ref, irel_ref, jl_ref, jlc_ref,
                 valid_ref, Wr2_ref, br2_ref, kcol_ref, out_ref, acc_ref, *, nf):
    p = pl.program_id(0)

    @pl.when(first_s[p] == 1)
    def _():
        acc_ref[...] = jnp.zeros_like(acc_ref)

    @pl.when(act_s[p] == 1)
    def _():
        irel = irel_ref[0]
        jl = jl_ref[0]
        jlc = jlc_ref[...]
        vrow = valid_ref[0]
        ios = jax.lax.broadcasted_iota(jnp.int32, (BLK, BLK), 0)
        iol = jax.lax.broadcasted_iota(jnp.int32, (BLK, BLK), 1)
        Gi1 = (ios == irel).astype(jnp.float32)
        Gi2 = (ios == (irel - BLK)).astype(jnp.float32)
        Gj = (ios == jl).astype(jnp.float32)
        GjT = (jlc == iol).astype(jnp.float32)
        hi = jax.lax.Precision.HIGHEST
        pi3 = (_mm(posw1_ref[...], Gi1, hi)
               + _mm(posw2_ref[...], Gi2, hi))
        pj3 = _mm(posj_ref[...], Gj, hi)
        xi, yi, zi = pi3[0:1, :], pi3[1:2, :], pi3[2:3, :]
        xj, yj, zj = pj3[0:1, :], pj3[1:2, :], pj3[2:3, :]
        dx = xi - xj
        dy = yi - yj
        dz = zi - zj
        d2m = ((xi * xi + yi * yi + zi * zi) + (xj * xj + yj * yj + zj * zj)
               - 2.0 * (xi * xj + yi * yj + zi * zj))
        dist = jnp.sqrt(jnp.maximum(d2m, 0.0))
        m = vrow * (dist < CUTOFF).astype(jnp.float32)
        nr = jnp.sqrt(dx * dx + dy * dy + dz * dz + 1e-12)
        inv = 1.0 / (nr + EPS)
        th = (jnp.pi / CUTOFF) * nr
        rbf = _sin_reduced(kcol_ref[...] * th) * inv
        lin = _mm(Wr2_ref[...], rbf, hi) + br2_ref[...]
        Wp = (0.5 * (_cos_clamped((jnp.pi / CUTOFF) * lin) + 1.0)
              * (lin < CUTOFF).astype(jnp.float32))
        W2 = Wp[0:nf, :]
        W3 = Wp[nf:2 * nf, :]
        tmp = jnp.concatenate(
            [W2 * m, W3 * (m * (dx * inv)), W3 * (m * (dy * inv)),
             W3 * (m * (dz * inv))], axis=0)
        acc_ref[...] += _mm(tmp, GjT, hi)

    @pl.when(last_s[p] == 1)
    def _():
        out_ref[...] = acc_ref[...]


def _update_body(accT_ref, phiT_ref,
                 WUx_ref, bUx_ref, WUy_ref, bUy_ref, WUz_ref, bUz_ref,
                 WVx_ref, bVx_ref, WVy_ref, bVy_ref, WVz_ref, bVz_ref,
                 Wm1_ref, bm1_ref, Wm2_ref, bm2_ref,
                 ds_ref, dv_ref, *, nf):
    phi2 = phiT_ref[0:nf, :]
    phi3 = phiT_ref[nf:2 * nf, :]
    s1T = phi2 * accT_ref[0:nf, :]
    v1x = phi3 * accT_ref[nf:2 * nf, :]
    v1y = phi3 * accT_ref[2 * nf:3 * nf, :]
    v1z = phi3 * accT_ref[3 * nf:4 * nf, :]
    Uvx = _mm(WUx_ref[...], v1x) + bUx_ref[...]
    Uvy = _mm(WUy_ref[...], v1y) + bUy_ref[...]
    Uvz = _mm(WUz_ref[...], v1z) + bUz_ref[...]
    Vvx = _mm(WVx_ref[...], v1x) + bVx_ref[...]
    Vvy = _mm(WVy_ref[...], v1y) + bVy_ref[...]
    Vvz = _mm(WVz_ref[...], v1z) + bVz_ref[...]
    Vn = jnp.sqrt(Vvx * Vvx + Vvy * Vvy + Vvz * Vvz + 1e-12)
    Wm1 = Wm1_ref[...]
    h = _mm(Wm1[:, 0:nf], Vn) + _mm(Wm1[:, nf:2 * nf], s1T) + bm1_ref[...]
    h = h * jax.nn.sigmoid(h)
    mlp = _mm(Wm2_ref[...], h) + bm2_ref[...]
    a_vv = mlp[0:nf, :]
    a_sv = mlp[nf:2 * nf, :]
    a_ss = mlp[2 * nf:3 * nf, :]
    ds_ref[...] = (Uvx * Vvx + Uvy * Vvy + Uvz * Vvz) * a_sv + a_ss
    dv_ref[0:nf, :] = a_vv * Uvx
    dv_ref[nf:2 * nf, :] = a_vv * Uvy
    dv_ref[2 * nf:3 * nf, :] = a_vv * Uvz


def kernel(atoms, atom_positions, graph_indexes, emb, Ws1, bs1, Ws2, bs2, Wr, br,
           Wm1, bm1, Wm2, bm2, WUx, bUx, WUy, bUy, WUz, bUz, WVx, bVx, WVy, bVy,
           WVz, bVz):
    n = atoms.shape[0]
    nf = emb.shape[1]
    num_rbf = Wr.shape[1]
    natype = emb.shape[0]
    atp = ((natype + BLK - 1) // BLK) * BLK
    npad = ((n + BLK - 1) // BLK) * BLK
    nb = npad // BLK
    npair = nb * nb

    f32 = jnp.float32
    pos = jnp.zeros((npad, 3), f32).at[:n].set(atom_positions.astype(f32))
    posT = pos.T
    gi = jnp.full((npad,), PAD_GI, jnp.int32).at[:n].set(
        graph_indexes.astype(jnp.int32))
    gcol = gi[:, None]
    grow = gi[None, :]
    atoms_row = jnp.zeros((1, npad), jnp.int32).at[0, :n].set(
        atoms.astype(jnp.int32))

    embT = jnp.zeros((nf, atp), f32).at[:, :natype].set(emb.T)
    Ws2b = Ws2[nf:3 * nf, :]
    bs2b = bs2[nf:3 * nf][:, None]
    Wr2 = Wr[nf:3 * nf, :]
    br2 = br[nf:3 * nf][:, None]

    phiT = pl.pallas_call(
        _phi_body,
        grid=(nb,),
        in_specs=[
            pl.BlockSpec((1, BLK), lambda b: (0, b)),
            pl.BlockSpec((nf, atp), lambda b: (0, 0)),
            pl.BlockSpec((nf, nf), lambda b: (0, 0)),
            pl.BlockSpec((nf, 1), lambda b: (0, 0)),
            pl.BlockSpec((2 * nf, nf), lambda b: (0, 0)),
            pl.BlockSpec((2 * nf, 1), lambda b: (0, 0)),
        ],
        out_specs=pl.BlockSpec((2 * nf, BLK), lambda b: (0, b)),
        out_shape=jax.ShapeDtypeStruct((2 * nf, npad), f32),
    )(atoms_row, embT, Ws1, bs1[:, None], Ws2b, bs2b)

    gb = gi.reshape(nb, BLK)
    glo = gb[:, 0]
    ghi = gb[:, -1]
    act_mat = (glo[:, None] <= ghi[None, :]) & (glo[None, :] <= ghi[:, None])
    flat = act_mat.T.reshape(-1)
    idx = jnp.nonzero(flat, size=npair, fill_value=0)[0]
    num = jnp.sum(flat.astype(jnp.int32))
    ar = jnp.arange(npair)
    lastidx = idx[num - 1]
    pidx = jnp.where(ar < num, idx, lastidx)
    bi_arr = (pidx % nb).astype(jnp.int32)
    bj_arr = (pidx // nb).astype(jnp.int32)
    act_arr = (ar < num).astype(jnp.int32)
    bj_prev = jnp.concatenate([jnp.array([-1], jnp.int32), bj_arr[:-1]])
    bj_next = jnp.concatenate([bj_arr[1:], jnp.array([-1], jnp.int32)])
    first_arr = ((bj_arr != bj_prev) & (ar < num)).astype(jnp.int32)
    last_arr = (((ar + 1 == num) | (bj_arr != bj_next)) & (ar < num)).astype(
        jnp.int32)

    grid_spec = pltpu.PrefetchScalarGridSpec(
        num_scalar_prefetch=5,
        grid=(npair,),
        in_specs=[
            pl.BlockSpec((BLK, 3), lambda p, bi, bj, f, l, a: (bi[p], 0)),
            pl.BlockSpec((3, BLK), lambda p, bi, bj, f, l, a: (0, bj[p])),
            pl.BlockSpec((BLK, 1), lambda p, bi, bj, f, l, a: (bi[p], 0)),
            pl.BlockSpec((1, BLK), lambda p, bi, bj, f, l, a: (0, bj[p])),
            pl.BlockSpec((2 * nf, num_rbf), lambda p, bi, bj, f, l, a: (0, 0)),
            pl.BlockSpec((2 * nf, 1), lambda p, bi, bj, f, l, a: (0, 0)),
            pl.BlockSpec((num_rbf, 1), lambda p, bi, bj, f, l, a: (0, 0)),
        ],
        out_specs=pl.BlockSpec((4 * nf, BLK), lambda p, bi, bj, f, l, a: (0, bj[p])),
        scratch_shapes=[pltpu.VMEM((4 * nf, BLK), f32),
                        pltpu.VMEM((6 * BLK, BLK), f32)],
    )
    accT_call = pl.pallas_call(
        functools.partial(_pair_body, nf=nf, num_rbf=num_rbf),
        grid_spec=grid_spec,
        out_shape=jax.ShapeDtypeStruct((4 * nf, npad), f32),
    )
    kcol = jnp.arange(1, num_rbf + 1, dtype=f32)[:, None]

    i32 = jnp.int32
    rcap = 4 * npad
    cap = npad // 2
    capl = cap * BLK
    idxn = jnp.arange(npad, dtype=i32)
    brk = gi[1:] != gi[:-1]
    startn = jnp.concatenate([jnp.ones(1, bool), brk])
    endn = jnp.concatenate([brk, jnp.ones(1, bool)])
    ms = jax.lax.cummax(jnp.where(startn, idxn, 0))
    me = jax.lax.cummin(jnp.where(endn, idxn + 1, npad), reverse=True)
    blk0 = jnp.arange(nb, dtype=i32) * BLK
    ilo = jnp.minimum(ms[blk0], n)
    ihi = jnp.minimum(me[blk0 + BLK - 1], n)
    cnt = jnp.maximum(ihi - ilo, 0)
    rstart = jnp.concatenate([jnp.zeros(1, i32),
                              jnp.cumsum(cnt, dtype=i32)])
    rtot = rstart[nb]
    r = jnp.arange(rcap, dtype=i32)
    rvalid = r < rtot
    rb = jnp.clip(jnp.sum((rstart[None, :nb] <= r[:, None]).astype(i32),
                          axis=1) - 1, 0, nb - 1)
    OH = (rb[:, None] == jnp.arange(nb, dtype=i32)[None, :]).astype(f32)
    T1 = jnp.stack([ilo.astype(f32), rstart[:nb].astype(f32)], axis=1)
    G1 = jax.lax.dot_general(OH, T1, (((1,), (0,)), ((), ())),
                             preferred_element_type=f32,
                             precision=jax.lax.Precision.HIGHEST)
    ri = jnp.clip(G1[:, 0].astype(i32) + (r - G1[:, 1].astype(i32)), 0, n - 1)
    rjs = jnp.maximum(jnp.take(ms, ri), rb * BLK)
    rje = jnp.minimum(jnp.take(me, ri), (rb + 1) * BLK)
    rL = jnp.where(rvalid, rje - rjs, 0)
    cumrl = jnp.concatenate([jnp.zeros(1, i32), jnp.cumsum(rL, dtype=i32)])
    cumat = jnp.take(cumrl, rstart)
    Sb = cumat[1:] - cumat[:nb]
    Spad = jnp.maximum(((Sb + BLK - 1) // BLK) * BLK, BLK)
    lanebase = jnp.concatenate([jnp.zeros(1, i32),
                                jnp.cumsum(Spad, dtype=i32)])
    chunkstart = lanebase // BLK
    ctot = chunkstart[nb]
    ttot = lanebase[nb]
    T2 = jnp.stack([lanebase[:nb].astype(f32), cumat[:nb].astype(f32)], axis=1)
    G2 = jax.lax.dot_general(OH, T2, (((1,), (0,)), ((), ())),
                             preferred_element_type=f32,
                             precision=jax.lax.Precision.HIGHEST)
    gls = jnp.where(rvalid,
                    G2[:, 0].astype(i32) + (cumrl[:rcap] - G2[:, 1].astype(i32)),
                    capl)
    lane = jnp.arange(capl, dtype=i32)

    def _dfill(vals, pos, ok):
        prev = jnp.concatenate([jnp.zeros(1, i32), vals[:-1]])
        delta = jnp.where(ok, vals - prev, 0)
        d = jnp.zeros((capl,), i32).at[jnp.minimum(pos, capl - 1)].add(delta)
        return jnp.cumsum(d, dtype=i32)

    li = jnp.clip(_dfill(ri, gls, rvalid), 0, n - 1)
    yv = _dfill(rjs - gls, gls, rvalid)
    bidx = jnp.arange(nb, dtype=i32)
    bok = jnp.ones((nb,), bool)
    lbf = _dfill(bidx, lanebase[:nb], bok)
    wf = _dfill(lanebase[:nb] + Sb, lanebase[:nb], bok)
    lvalid = lane < wf
    ljl = jnp.clip((yv + lane) - lbf * BLK, 0, BLK - 1)
    liM = li.reshape(cap, BLK)
    lvM = lvalid.reshape(cap, BLK)
    minli = jnp.min(jnp.where(lvM, liM, 1 << 30), axis=1)
    iw_c = jnp.clip(minli // BLK, 0, nb - 1).astype(i32)
    irelM = jnp.clip(liM - iw_c[:, None] * BLK, 0, 2 * BLK - 1).astype(i32)
    c = jnp.arange(cap, dtype=i32)
    cb = jnp.clip(jnp.sum((chunkstart[None, :nb] <= c[:, None]).astype(i32),
                          axis=1) - 1, 0, nb - 1)
    actc = c < ctot
    cs_cb = jnp.take(chunkstart, cb)
    cs_cb1 = jnp.take(chunkstart, cb + 1)
    firstc = (actc & (c == cs_cb)).astype(i32)
    lastc = (actc & (c + 1 == cs_cb1)).astype(i32)
    lastreal = jnp.maximum(ctot - 1, 0)
    iw_c = jnp.where(actc, iw_c, iw_c[lastreal]).astype(i32)
    cm_c = jnp.minimum(c, lastreal).astype(i32)
    actc_i = actc.astype(i32)
    okpack = (rtot <= rcap) & (lanebase[nb] <= capl)

    posg = jnp.zeros((3, (nb + 1) * BLK), f32).at[:, :npad].set(posT)
    irel3 = irelM.reshape(cap, 1, BLK)
    jl3 = ljl.reshape(cap, 1, BLK)
    jlc2 = ljl.reshape(capl, 1)
    valid3 = lvM.astype(f32).reshape(cap, 1, BLK)

    packed_spec = pltpu.PrefetchScalarGridSpec(
        num_scalar_prefetch=6,
        grid=(cap,),
        in_specs=[
            pl.BlockSpec((3, BLK), lambda p, bj, iw, fi, la, ac, cm: (0, iw[p])),
            pl.BlockSpec((3, BLK), lambda p, bj, iw, fi, la, ac, cm: (0, iw[p] + 1)),
            pl.BlockSpec((3, BLK), lambda p, bj, iw, fi, la, ac, cm: (0, bj[p])),
            pl.BlockSpec((1, 1, BLK), lambda p, bj, iw, fi, la, ac, cm: (cm[p], 0, 0)),
            pl.BlockSpec((1, 1, BLK), lambda p, bj, iw, fi, la, ac, cm: (cm[p], 0, 0)),
            pl.BlockSpec((BLK, 1), lambda p, bj, iw, fi, la, ac, cm: (cm[p], 0)),
            pl.BlockSpec((1, 1, BLK), lambda p, bj, iw, fi, la, ac, cm: (cm[p], 0, 0)),
            pl.BlockSpec((2 * nf, num_rbf), lambda p, bj, iw, fi, la, ac, cm: (0, 0)),
            pl.BlockSpec((2 * nf, 1), lambda p, bj, iw, fi, la, ac, cm: (0, 0)),
            pl.BlockSpec((num_rbf, 1), lambda p, bj, iw, fi, la, ac, cm: (0, 0)),
        ],
        out_specs=pl.BlockSpec((4 * nf, BLK),
                               lambda p, bj, iw, fi, la, ac, cm: (0, bj[p])),
        scratch_shapes=[pltpu.VMEM((4 * nf, BLK), f32)],
    )
    packed_call = pl.pallas_call(
        functools.partial(_packed_body, nf=nf),
        grid_spec=packed_spec,
        out_shape=jax.ShapeDtypeStruct((4 * nf, npad), f32),
    )

    def _run_packed(_):
        return packed_call(bj_arr_p, iw_c, firstc, lastc, actc_i, cm_c,
                           posg, posg, posg, irel3, jl3, jlc2, valid3,
                           Wr2, br2, kcol)

    def _run_tiles(_):
        return accT_call(bi_arr, bj_arr, first_arr, last_arr, act_arr,
                         pos, posT, gcol, grow, Wr2, br2, kcol)

    bj_arr_p = cb
    accT = jax.lax.cond(okpack, _run_packed, _run_tiles, None)

    wspec = pl.BlockSpec((nf, nf), lambda b: (0, 0))
    bspec = pl.BlockSpec((nf, 1), lambda b: (0, 0))
    dsT, dvT = pl.pallas_call(
        functools.partial(_update_body, nf=nf),
        grid=(nb,),
        in_specs=[
            pl.BlockSpec((4 * nf, BLK), lambda b: (0, b)),
            pl.BlockSpec((2 * nf, BLK), lambda b: (0, b)),
            wspec, bspec, wspec, bspec, wspec, bspec,
            wspec, bspec, wspec, bspec, wspec, bspec,
            pl.BlockSpec((nf, 2 * nf), lambda b: (0, 0)),
            bspec,
            pl.BlockSpec((3 * nf, nf), lambda b: (0, 0)),
            pl.BlockSpec((3 * nf, 1), lambda b: (0, 0)),
        ],
        out_specs=[
            pl.BlockSpec((nf, BLK), lambda b: (0, b)),
            pl.BlockSpec((3 * nf, BLK), lambda b: (0, b)),
        ],
        out_shape=[
            jax.ShapeDtypeStruct((nf, npad), f32),
            jax.ShapeDtypeStruct((3 * nf, npad), f32),
        ],
    )(accT, phiT,
      WUx, bUx[:, None], WUy, bUy[:, None], WUz, bUz[:, None],
      WVx, bVx[:, None], WVy, bVy[:, None], WVz, bVz[:, None],
      Wm1, bm1[:, None], Wm2, bm2[:, None])

    delta_s = dsT.T[:n]
    delta_v = jnp.stack([dvT[0:nf, :].T[:n], dvT[nf:2 * nf, :].T[:n],
                         dvT[2 * nf:3 * nf, :].T[:n]], axis=2)
    return delta_s, delta_v


# speedup vs baseline: 68.4217x; 1.6861x over previous
import functools

import jax
import jax.numpy as jnp
from jax.experimental import pallas as pl
from jax.experimental.pallas import tpu as pltpu

BLK = 128
CUTOFF = 5.0
EPS = 1e-8
PAD_GI = 10**9


_COS_C = (9.9999999228e-01, -4.9999991772e-01, 4.1666524364e-02,
          -1.3887970411e-03, 2.4773424205e-05, -2.7113373272e-07,
          1.7369133674e-09)
_SIN_C = (9.9999970703e-01, -1.6666577217e-01, 8.3325581326e-03,
          -1.9812575931e-04, 2.7040516974e-06, -2.0534265044e-08)
_TWO_PI_HI = 6.28125
_TWO_PI_LO = 1.9353071795864768e-03
_INV_TWO_PI = 0.15915494309189535


def _cos_clamped(y):
    y = jnp.clip(y, -jnp.pi, jnp.pi)
    z = y * y
    c = _COS_C
    r = c[6]
    for k in (5, 4, 3, 2, 1, 0):
        r = r * z + c[k]
    return r


def _sin_reduced(a):
    q = jnp.floor(a * _INV_TWO_PI + 0.5)
    x = (a - q * _TWO_PI_HI) - q * _TWO_PI_LO
    z = x * x
    s = _SIN_C
    r = s[5]
    for k in (4, 3, 2, 1, 0):
        r = r * z + s[k]
    return x * r


def _phi_body(atoms_ref, embT_ref, Ws1_ref, bs1_ref, Ws2b_ref, bs2b_ref, phiT_ref):
    atp = embT_ref.shape[1]
    arow = atoms_ref[...]
    ioa = jax.lax.broadcasted_iota(jnp.int32, (atp, BLK), 0)
    oh = (ioa == arow).astype(jnp.float32)
    s0T = jax.lax.dot_general(embT_ref[...], oh, (((1,), (0,)), ((), ())),
                              preferred_element_type=jnp.float32)
    h = jax.lax.dot_general(Ws1_ref[...], s0T, (((1,), (0,)), ((), ())),
                            preferred_element_type=jnp.float32) + bs1_ref[...]
    h = h * jax.nn.sigmoid(h)
    phiT_ref[...] = jax.lax.dot_general(Ws2b_ref[...], h, (((1,), (0,)), ((), ())),
                                        preferred_element_type=jnp.float32) + bs2b_ref[...]


def _pair_body(bi_s, bj_s, first_s, last_s, act_s,
               pos_i_ref, posT_j_ref, gcol_ref, grow_ref, Wr2_ref, br2_ref,
               kcol_ref, out_ref, acc_ref, tmp_ref, *, nf, num_rbf):
    p = pl.program_id(0)

    @pl.when(first_s[p] == 1)
    def _():
        acc_ref[...] = jnp.zeros_like(acc_ref)

    @pl.when(act_s[p] == 1)
    def _():
        pos_i = pos_i_ref[...]
        posTj = posT_j_ref[...]
        xi, yi, zi = pos_i[:, 0:1], pos_i[:, 1:2], pos_i[:, 2:3]
        xj, yj, zj = posTj[0:1, :], posTj[1:2, :], posTj[2:3, :]
        g = xi * xj + yi * yj + zi * zj
        pn_i = xi * xi + yi * yi + zi * zi
        pn_j = xj * xj + yj * yj + zj * zj
        d2 = jnp.maximum(pn_i + pn_j - 2.0 * g, 0.0)
        dist = jnp.sqrt(d2)
        m = jnp.logical_and(gcol_ref[...] == grow_ref[...],
                            dist < CUTOFF).astype(jnp.float32)
        nr = jnp.sqrt(d2 + 1e-12)
        inv = 1.0 / (nr + EPS)
        tmp_ref[0 * BLK:1 * BLK, :] = (jnp.pi / CUTOFF) * nr
        tmp_ref[1 * BLK:2 * BLK, :] = inv
        tmp_ref[2 * BLK:3 * BLK, :] = m
        tmp_ref[3 * BLK:4 * BLK, :] = m * ((xi - xj) * inv)
        tmp_ref[4 * BLK:5 * BLK, :] = m * ((yi - yj) * inv)
        tmp_ref[5 * BLK:6 * BLK, :] = m * ((zi - zj) * inv)
        rowact = jnp.sum(m, axis=1, keepdims=True)
        idxc = jax.lax.broadcasted_iota(jnp.int32, (BLK, 1), 0)
        lo = jnp.min(jnp.where(rowact > 0.0, idxc, BLK))
        hi = jnp.max(jnp.where(rowact > 0.0, idxc, -1))
        kvec = kcol_ref[...]
        Wr2 = Wr2_ref[...]
        br2 = br2_ref[...]

        def body(i, carry):
            th = tmp_ref[pl.ds(0 * BLK + i, 1), :]
            iv = tmp_ref[pl.ds(1 * BLK + i, 1), :]
            rbf = _sin_reduced(kvec * th) * iv
            lin = jax.lax.dot_general(Wr2, rbf, (((1,), (0,)), ((), ())),
                                      preferred_element_type=jnp.float32) + br2
            Wp = (0.5 * (_cos_clamped((jnp.pi / CUTOFF) * lin) + 1.0)
                  * (lin < CUTOFF).astype(jnp.float32))
            W2 = Wp[0:nf, :]
            W3 = Wp[nf:2 * nf, :]
            acc_ref[0:nf, :] += W2 * tmp_ref[pl.ds(2 * BLK + i, 1), :]
            acc_ref[nf:2 * nf, :] += W3 * tmp_ref[pl.ds(3 * BLK + i, 1), :]
            acc_ref[2 * nf:3 * nf, :] += W3 * tmp_ref[pl.ds(4 * BLK + i, 1), :]
            acc_ref[3 * nf:4 * nf, :] += W3 * tmp_ref[pl.ds(5 * BLK + i, 1), :]
            return carry

        jax.lax.fori_loop(lo, hi + 1, body, 0)

    @pl.when(last_s[p] == 1)
    def _():
        out_ref[...] = acc_ref[...]


def _mm(a, b, precision=None):
    return jax.lax.dot_general(a, b, (((1,), (0,)), ((), ())),
                               preferred_element_type=jnp.float32,
                               precision=precision)


def _packed_body(bj_s, iw_s, first_s, last_s, act_s, cm_s,
                 posw1_ref, posw2_ref, posj_ref, meta_ref,
                 Wr2_ref, br2_ref, kcol_ref, out_ref, acc_ref, *, nf):
    p = pl.program_id(0)

    @pl.when(first_s[p] == 1)
    def _():
        acc_ref[...] = jnp.zeros_like(acc_ref)

    @pl.when(act_s[p] == 1)
    def _():
        meta = meta_ref[0]
        irel = meta[0:1, :]
        jl = meta[1:2, :]
        vrow = meta[2:3, :].astype(jnp.float32)
        ios = jax.lax.broadcasted_iota(jnp.int32, (BLK, BLK), 0)
        Gi1 = (ios == irel).astype(jnp.float32)
        Gi2 = (ios == (irel - BLK)).astype(jnp.float32)
        Gj = (ios == jl).astype(jnp.float32)
        hi = jax.lax.Precision.HIGHEST
        pi3 = (_mm(posw1_ref[...], Gi1, hi)
               + _mm(posw2_ref[...], Gi2, hi))
        pj3 = _mm(posj_ref[...], Gj, hi)
        xi, yi, zi = pi3[0:1, :], pi3[1:2, :], pi3[2:3, :]
        xj, yj, zj = pj3[0:1, :], pj3[1:2, :], pj3[2:3, :]
        dx = xi - xj
        dy = yi - yj
        dz = zi - zj
        d2m = ((xi * xi + yi * yi + zi * zi) + (xj * xj + yj * yj + zj * zj)
               - 2.0 * (xi * xj + yi * yj + zi * zj))
        dist = jnp.sqrt(jnp.maximum(d2m, 0.0))
        m = vrow * (dist < CUTOFF).astype(jnp.float32)
        nr = jnp.sqrt(dx * dx + dy * dy + dz * dz + 1e-12)
        inv = 1.0 / (nr + EPS)
        th = (jnp.pi / CUTOFF) * nr
        rbf = _sin_reduced(kcol_ref[...] * th) * inv
        lin = _mm(Wr2_ref[...], rbf, hi) + br2_ref[...]
        Wp = (0.5 * (_cos_clamped((jnp.pi / CUTOFF) * lin) + 1.0)
              * (lin < CUTOFF).astype(jnp.float32))
        W2 = Wp[0:nf, :]
        W3 = Wp[nf:2 * nf, :]
        tmp = jnp.concatenate(
            [W2 * m, W3 * (m * (dx * inv)), W3 * (m * (dy * inv)),
             W3 * (m * (dz * inv))], axis=0)
        acc_ref[...] += jax.lax.dot_general(
            tmp, Gj, (((1,), (1,)), ((), ())),
            preferred_element_type=jnp.float32,
            precision=hi)

    @pl.when(last_s[p] == 1)
    def _():
        out_ref[...] = acc_ref[...]


def _update_body(accT_ref, phiT_ref,
                 WUx_ref, bUx_ref, WUy_ref, bUy_ref, WUz_ref, bUz_ref,
                 WVx_ref, bVx_ref, WVy_ref, bVy_ref, WVz_ref, bVz_ref,
                 Wm1_ref, bm1_ref, Wm2_ref, bm2_ref,
                 ds_ref, dv_ref, *, nf):
    phi2 = phiT_ref[0:nf, :]
    phi3 = phiT_ref[nf:2 * nf, :]
    s1T = phi2 * accT_ref[0:nf, :]
    v1x = phi3 * accT_ref[nf:2 * nf, :]
    v1y = phi3 * accT_ref[2 * nf:3 * nf, :]
    v1z = phi3 * accT_ref[3 * nf:4 * nf, :]
    Uvx = _mm(WUx_ref[...], v1x) + bUx_ref[...]
    Uvy = _mm(WUy_ref[...], v1y) + bUy_ref[...]
    Uvz = _mm(WUz_ref[...], v1z) + bUz_ref[...]
    Vvx = _mm(WVx_ref[...], v1x) + bVx_ref[...]
    Vvy = _mm(WVy_ref[...], v1y) + bVy_ref[...]
    Vvz = _mm(WVz_ref[...], v1z) + bVz_ref[...]
    Vn = jnp.sqrt(Vvx * Vvx + Vvy * Vvy + Vvz * Vvz + 1e-12)
    Wm1 = Wm1_ref[...]
    h = _mm(Wm1[:, 0:nf], Vn) + _mm(Wm1[:, nf:2 * nf], s1T) + bm1_ref[...]
    h = h * jax.nn.sigmoid(h)
    mlp = _mm(Wm2_ref[...], h) + bm2_ref[...]
    a_vv = mlp[0:nf, :]
    a_sv = mlp[nf:2 * nf, :]
    a_ss = mlp[2 * nf:3 * nf, :]
    ds_ref[...] = (Uvx * Vvx + Uvy * Vvy + Uvz * Vvz) * a_sv + a_ss
    dv_ref[0:nf, :] = a_vv * Uvx
    dv_ref[nf:2 * nf, :] = a_vv * Uvy
    dv_ref[2 * nf:3 * nf, :] = a_vv * Uvz


def kernel(atoms, atom_positions, graph_indexes, emb, Ws1, bs1, Ws2, bs2, Wr, br,
           Wm1, bm1, Wm2, bm2, WUx, bUx, WUy, bUy, WUz, bUz, WVx, bVx, WVy, bVy,
           WVz, bVz):
    n = atoms.shape[0]
    nf = emb.shape[1]
    num_rbf = Wr.shape[1]
    natype = emb.shape[0]
    atp = ((natype + BLK - 1) // BLK) * BLK
    npad = ((n + BLK - 1) // BLK) * BLK
    nb = npad // BLK
    npair = nb * nb

    f32 = jnp.float32
    pos = jnp.zeros((npad, 3), f32).at[:n].set(atom_positions.astype(f32))
    posT = pos.T
    gi = jnp.full((npad,), PAD_GI, jnp.int32).at[:n].set(
        graph_indexes.astype(jnp.int32))
    gcol = gi[:, None]
    grow = gi[None, :]
    atoms_row = jnp.zeros((1, npad), jnp.int32).at[0, :n].set(
        atoms.astype(jnp.int32))

    embT = jnp.zeros((nf, atp), f32).at[:, :natype].set(emb.T)
    Ws2b = Ws2[nf:3 * nf, :]
    bs2b = bs2[nf:3 * nf][:, None]
    Wr2 = Wr[nf:3 * nf, :]
    br2 = br[nf:3 * nf][:, None]

    phiT = pl.pallas_call(
        _phi_body,
        grid=(nb,),
        in_specs=[
            pl.BlockSpec((1, BLK), lambda b: (0, b)),
            pl.BlockSpec((nf, atp), lambda b: (0, 0)),
            pl.BlockSpec((nf, nf), lambda b: (0, 0)),
            pl.BlockSpec((nf, 1), lambda b: (0, 0)),
            pl.BlockSpec((2 * nf, nf), lambda b: (0, 0)),
            pl.BlockSpec((2 * nf, 1), lambda b: (0, 0)),
        ],
        out_specs=pl.BlockSpec((2 * nf, BLK), lambda b: (0, b)),
        out_shape=jax.ShapeDtypeStruct((2 * nf, npad), f32),
    )(atoms_row, embT, Ws1, bs1[:, None], Ws2b, bs2b)

    gb = gi.reshape(nb, BLK)
    glo = gb[:, 0]
    ghi = gb[:, -1]
    act_mat = (glo[:, None] <= ghi[None, :]) & (glo[None, :] <= ghi[:, None])
    flat = act_mat.T.reshape(-1)
    idx = jnp.nonzero(flat, size=npair, fill_value=0)[0]
    num = jnp.sum(flat.astype(jnp.int32))
    ar = jnp.arange(npair)
    lastidx = idx[num - 1]
    pidx = jnp.where(ar < num, idx, lastidx)
    bi_arr = (pidx % nb).astype(jnp.int32)
    bj_arr = (pidx // nb).astype(jnp.int32)
    act_arr = (ar < num).astype(jnp.int32)
    bj_prev = jnp.concatenate([jnp.array([-1], jnp.int32), bj_arr[:-1]])
    bj_next = jnp.concatenate([bj_arr[1:], jnp.array([-1], jnp.int32)])
    first_arr = ((bj_arr != bj_prev) & (ar < num)).astype(jnp.int32)
    last_arr = (((ar + 1 == num) | (bj_arr != bj_next)) & (ar < num)).astype(
        jnp.int32)

    grid_spec = pltpu.PrefetchScalarGridSpec(
        num_scalar_prefetch=5,
        grid=(npair,),
        in_specs=[
            pl.BlockSpec((BLK, 3), lambda p, bi, bj, f, l, a: (bi[p], 0)),
            pl.BlockSpec((3, BLK), lambda p, bi, bj, f, l, a: (0, bj[p])),
            pl.BlockSpec((BLK, 1), lambda p, bi, bj, f, l, a: (bi[p], 0)),
            pl.BlockSpec((1, BLK), lambda p, bi, bj, f, l, a: (0, bj[p])),
            pl.BlockSpec((2 * nf, num_rbf), lambda p, bi, bj, f, l, a: (0, 0)),
            pl.BlockSpec((2 * nf, 1), lambda p, bi, bj, f, l, a: (0, 0)),
            pl.BlockSpec((num_rbf, 1), lambda p, bi, bj, f, l, a: (0, 0)),
        ],
        out_specs=pl.BlockSpec((4 * nf, BLK), lambda p, bi, bj, f, l, a: (0, bj[p])),
        scratch_shapes=[pltpu.VMEM((4 * nf, BLK), f32),
                        pltpu.VMEM((6 * BLK, BLK), f32)],
    )
    accT_call = pl.pallas_call(
        functools.partial(_pair_body, nf=nf, num_rbf=num_rbf),
        grid_spec=grid_spec,
        out_shape=jax.ShapeDtypeStruct((4 * nf, npad), f32),
    )
    kcol = jnp.arange(1, num_rbf + 1, dtype=f32)[:, None]

    i32 = jnp.int32
    rcap = 2 * npad
    cap = npad // 4
    capl = cap * BLK
    idxn = jnp.arange(npad, dtype=i32)
    brk = gi[1:] != gi[:-1]
    startn = jnp.concatenate([jnp.ones(1, bool), brk])
    endn = jnp.concatenate([brk, jnp.ones(1, bool)])
    ms = jax.lax.cummax(jnp.where(startn, idxn, 0))
    me = jax.lax.cummin(jnp.where(endn, idxn + 1, npad), reverse=True)
    blk0 = jnp.arange(nb, dtype=i32) * BLK
    ilo = jnp.minimum(ms[blk0], n)
    ihi = jnp.minimum(me[blk0 + BLK - 1], n)
    cnt = jnp.maximum(ihi - ilo, 0)
    rstart = jnp.concatenate([jnp.zeros(1, i32),
                              jnp.cumsum(cnt, dtype=i32)])
    rtot = rstart[nb]
    r = jnp.arange(rcap, dtype=i32)
    rvalid = r < rtot
    rb = jnp.clip(jnp.sum((rstart[None, :nb] <= r[:, None]).astype(i32),
                          axis=1) - 1, 0, nb - 1)
    OH = (rb[:, None] == jnp.arange(nb, dtype=i32)[None, :]).astype(f32)
    T1 = jnp.stack([ilo.astype(f32), rstart[:nb].astype(f32)], axis=1)
    G1 = jax.lax.dot_general(OH, T1, (((1,), (0,)), ((), ())),
                             preferred_element_type=f32,
                             precision=jax.lax.Precision.HIGHEST)
    ri = jnp.clip(G1[:, 0].astype(i32) + (r - G1[:, 1].astype(i32)), 0, n - 1)
    rjs = jnp.maximum(jnp.take(ms, ri), rb * BLK)
    rje = jnp.minimum(jnp.take(me, ri), (rb + 1) * BLK)
    rL = jnp.where(rvalid, rje - rjs, 0)
    cumrl = jnp.concatenate([jnp.zeros(1, i32), jnp.cumsum(rL, dtype=i32)])
    cumat = jnp.take(cumrl, rstart)
    Sb = cumat[1:] - cumat[:nb]
    Spad = jnp.maximum(((Sb + BLK - 1) // BLK) * BLK, BLK)
    lanebase = jnp.concatenate([jnp.zeros(1, i32),
                                jnp.cumsum(Spad, dtype=i32)])
    chunkstart = lanebase // BLK
    ctot = chunkstart[nb]
    ttot = lanebase[nb]
    T2 = jnp.stack([lanebase[:nb].astype(f32), cumat[:nb].astype(f32)], axis=1)
    G2 = jax.lax.dot_general(OH, T2, (((1,), (0,)), ((), ())),
                             preferred_element_type=f32,
                             precision=jax.lax.Precision.HIGHEST)
    gls = jnp.where(rvalid,
                    G2[:, 0].astype(i32) + (cumrl[:rcap] - G2[:, 1].astype(i32)),
                    capl)
    lane = jnp.arange(capl, dtype=i32)

    def _dfill(vals, pos, ok):
        prev = jnp.concatenate([jnp.zeros(1, i32), vals[:-1]])
        delta = jnp.where(ok, vals - prev, 0)
        d = jnp.zeros((capl,), i32).at[jnp.minimum(pos, capl - 1)].add(delta)
        return jnp.cumsum(d, dtype=i32)

    li = jnp.clip(_dfill(ri, gls, rvalid), 0, n - 1)
    yv = _dfill(rjs - gls, gls, rvalid)
    bidx = jnp.arange(nb, dtype=i32)
    bok = jnp.ones((nb,), bool)
    lbf = _dfill(bidx, lanebase[:nb], bok)
    wf = _dfill(lanebase[:nb] + Sb, lanebase[:nb], bok)
    lvalid = lane < wf
    ljl = jnp.clip((yv + lane) - lbf * BLK, 0, BLK - 1)
    liM = li.reshape(cap, BLK)
    lvM = lvalid.reshape(cap, BLK)
    minli = jnp.min(jnp.where(lvM, liM, 1 << 30), axis=1)
    iw_c = jnp.clip(minli // BLK, 0, nb - 1).astype(i32)
    irelM = jnp.clip(liM - iw_c[:, None] * BLK, 0, 2 * BLK - 1).astype(i32)
    c = jnp.arange(cap, dtype=i32)
    cb = jnp.clip(jnp.sum((chunkstart[None, :nb] <= c[:, None]).astype(i32),
                          axis=1) - 1, 0, nb - 1)
    actc = c < ctot
    cs_cb = jnp.take(chunkstart, cb)
    cs_cb1 = jnp.take(chunkstart, cb + 1)
    firstc = (actc & (c == cs_cb)).astype(i32)
    lastc = (actc & (c + 1 == cs_cb1)).astype(i32)
    lastreal = jnp.maximum(ctot - 1, 0)
    iw_c = jnp.where(actc, iw_c, iw_c[lastreal]).astype(i32)
    cm_c = jnp.minimum(c, lastreal).astype(i32)
    actc_i = actc.astype(i32)
    okpack = (rtot <= rcap) & (lanebase[nb] <= capl)

    posg = jnp.zeros((3, (nb + 1) * BLK), f32).at[:, :npad].set(posT)
    meta = jnp.zeros((cap, 8, BLK), i32)
    meta = meta.at[:, 0, :].set(irelM)
    meta = meta.at[:, 1, :].set(ljl.reshape(cap, BLK))
    meta = meta.at[:, 2, :].set(lvM.astype(i32))

    packed_spec = pltpu.PrefetchScalarGridSpec(
        num_scalar_prefetch=6,
        grid=(cap,),
        in_specs=[
            pl.BlockSpec((3, BLK), lambda p, bj, iw, fi, la, ac, cm: (0, iw[p])),
            pl.BlockSpec((3, BLK), lambda p, bj, iw, fi, la, ac, cm: (0, iw[p] + 1)),
            pl.BlockSpec((3, BLK), lambda p, bj, iw, fi, la, ac, cm: (0, bj[p])),
            pl.BlockSpec((1, 8, BLK), lambda p, bj, iw, fi, la, ac, cm: (cm[p], 0, 0)),
            pl.BlockSpec((2 * nf, num_rbf), lambda p, bj, iw, fi, la, ac, cm: (0, 0)),
            pl.BlockSpec((2 * nf, 1), lambda p, bj, iw, fi, la, ac, cm: (0, 0)),
            pl.BlockSpec((num_rbf, 1), lambda p, bj, iw, fi, la, ac, cm: (0, 0)),
        ],
        out_specs=pl.BlockSpec((4 * nf, BLK),
                               lambda p, bj, iw, fi, la, ac, cm: (0, bj[p])),
        scratch_shapes=[pltpu.VMEM((4 * nf, BLK), f32)],
    )
    packed_call = pl.pallas_call(
        functools.partial(_packed_body, nf=nf),
        grid_spec=packed_spec,
        out_shape=jax.ShapeDtypeStruct((4 * nf, npad), f32),
    )

    def _run_packed(_):
        return packed_call(bj_arr_p, iw_c, firstc, lastc, actc_i, cm_c,
                           posg, posg, posg, meta, Wr2, br2, kcol)

    def _run_tiles(_):
        return accT_call(bi_arr, bj_arr, first_arr, last_arr, act_arr,
                         pos, posT, gcol, grow, Wr2, br2, kcol)

    bj_arr_p = cb
    accT = jax.lax.cond(okpack, _run_packed, _run_tiles, None)

    wspec = pl.BlockSpec((nf, nf), lambda b: (0, 0))
    bspec = pl.BlockSpec((nf, 1), lambda b: (0, 0))
    dsT, dvT = pl.pallas_call(
        functools.partial(_update_body, nf=nf),
        grid=(nb,),
        in_specs=[
            pl.BlockSpec((4 * nf, BLK), lambda b: (0, b)),
            pl.BlockSpec((2 * nf, BLK), lambda b: (0, b)),
            wspec, bspec, wspec, bspec, wspec, bspec,
            wspec, bspec, wspec, bspec, wspec, bspec,
            pl.BlockSpec((nf, 2 * nf), lambda b: (0, 0)),
            bspec,
            pl.BlockSpec((3 * nf, nf), lambda b: (0, 0)),
            pl.BlockSpec((3 * nf, 1), lambda b: (0, 0)),
        ],
        out_specs=[
            pl.BlockSpec((nf, BLK), lambda b: (0, b)),
            pl.BlockSpec((3 * nf, BLK), lambda b: (0, b)),
        ],
        out_shape=[
            jax.ShapeDtypeStruct((nf, npad), f32),
            jax.ShapeDtypeStruct((3 * nf, npad), f32),
        ],
    )(accT, phiT,
      WUx, bUx[:, None], WUy, bUy[:, None], WUz, bUz[:, None],
      WVx, bVx[:, None], WVy, bVy[:, None], WVz, bVz[:, None],
      Wm1, bm1[:, None], Wm2, bm2[:, None])

    delta_s = dsT.T[:n]
    delta_v = jnp.stack([dvT[0:nf, :].T[:n], dvT[nf:2 * nf, :].T[:n],
                         dvT[2 * nf:3 * nf, :].T[:n]], axis=2)
    return delta_s, delta_v
